# baseline (speedup 1.0000x reference)
import functools

import jax
import jax.numpy as jnp
from jax import lax
from jax.experimental import pallas as pl
from jax.experimental.pallas import tpu as pltpu
from jax.experimental.pallas import tpu_sc as plsc

_LANES = 128


def _knn_body(k_neigh, cc, xall_ref, xblk_ref, w1_ref, idx_ref, p_ref, s_ref,
              sq_ref):
    b = pl.program_id(0)
    rblk = pl.program_id(1)
    xblk = xblk_ref[0]
    r, n = s_ref.shape
    n_cc = n // cc
    row_g = rblk * r + lax.broadcasted_iota(jnp.int32, (r, cc), 0)
    inf = jnp.float32(jnp.inf)

    @pl.when(rblk == 0)
    def _():
        for j in range(n_cc):
            xc = xall_ref[0, pl.ds(j * cc, cc), :]
            sq_ref[pl.ds(j * cc, cc), :] = jnp.sum(xc * xc, axis=1, keepdims=True)

    xblk2 = -2.0 * xblk
    ones_col = jnp.ones((r, 1), jnp.float32)

    for j in range(n_cc):
        xc = xall_ref[0, pl.ds(j * cc, cc), :]
        sq = sq_ref[pl.ds(j * cc, cc), :]
        dot = lax.dot_general(
            xblk2, xc, (((1,), (1,)), ((), ())), preferred_element_type=jnp.float32
        )
        bcast = lax.dot_general(
            ones_col, sq, (((1,), (1,)), ((), ())),
            preferred_element_type=jnp.float32,
            precision=lax.Precision.HIGHEST,
        )
        sv = bcast + dot
        col = j * cc + lax.broadcasted_iota(jnp.int32, (r, cc), 1)
        s_ref[:, pl.ds(j * cc, cc)] = jnp.where(col == row_g, inf, sv)

    lane = lax.broadcasted_iota(jnp.int32, (r, _LANES), 1)
    base = b * n
    lanes_per_chunk = cc // _LANES

    def step(t, carry):
        amin_prev, acc = carry

        run_v = jnp.full((r, _LANES), inf)
        run_c = jnp.full((r, _LANES), n, jnp.int32)
        for j in range(n_cc):
            sv = s_ref[:, pl.ds(j * cc, cc)]
            col = j * cc + lax.broadcasted_iota(jnp.int32, (r, cc), 1)
            sv = jnp.where(col == amin_prev, inf, sv)
            s_ref[:, pl.ds(j * cc, cc)] = sv
            sv4 = sv.reshape(r, lanes_per_chunk, _LANES)
            for g in range(lanes_per_chunk):
                svg = sv4[:, g, :]
                colg = (j * cc + g * _LANES) + lane
                take = svg < run_v
                run_v = jnp.where(take, svg, run_v)
                run_c = jnp.where(take, colg, run_c)
        m = jnp.min(run_v, axis=1, keepdims=True)
        amin = jnp.min(
            jnp.where(run_v <= m, run_c, n), axis=1, keepdims=True
        )
        return amin, jnp.where(lane == t, amin + base, acc)

    amin0 = jnp.full((r, 1), -1, jnp.int32)
    _, acc = lax.fori_loop(
        0, k_neigh, step, (amin0, jnp.zeros((r, _LANES), jnp.int32))
    )
    idx_ref[0] = acc
    p_ref[0] = jnp.dot(xblk, w1_ref[...], preferred_element_type=jnp.float32)


def _knn(pos, w1, k_neigh, row_blk, col_chunk=512):
    b, n, c = pos.shape
    grid = (b, n // row_blk)
    return pl.pallas_call(
        functools.partial(_knn_body, k_neigh, col_chunk),
        grid=grid,
        in_specs=[
            pl.BlockSpec((1, n, c), lambda i, j: (i, 0, 0)),
            pl.BlockSpec((1, row_blk, c), lambda i, j: (i, j, 0)),
            pl.BlockSpec((c, c), lambda i, j: (0, 0)),
        ],
        out_specs=[
            pl.BlockSpec((1, row_blk, _LANES), lambda i, j: (i, j, 0)),
            pl.BlockSpec((1, row_blk, c), lambda i, j: (i, j, 0)),
        ],
        out_shape=[
            jax.ShapeDtypeStruct((b, n, _LANES), jnp.int32),
            jax.ShapeDtypeStruct((b, n, c), jnp.float32),
        ],
        scratch_shapes=[
            pltpu.VMEM((row_blk, n), jnp.float32),
            pltpu.VMEM((n, 1), jnp.float32),
        ],
    )(pos, pos, w1)


def _make_sc_gather(n_idx, d, chunk):
    info = plsc.get_sparse_core_info()
    nw = info.num_cores * info.num_subcores
    per_w = n_idx // nw
    n_chunks = per_w // chunk
    mesh = plsc.VectorSubcoreMesh(core_axis_name="c", subcore_axis_name="s")

    @functools.partial(
        pl.kernel,
        out_type=jax.ShapeDtypeStruct((n_idx, d), jnp.float32),
        mesh=mesh,
        scratch_types=[
            pltpu.VMEM((chunk,), jnp.int32),
            pltpu.VMEM((chunk, d), jnp.float32),
            pltpu.SemaphoreType.DMA,
        ],
    )
    def k(table_hbm, idx_hbm, out_hbm, idx_v, rows_v, sem):
        wid = lax.axis_index("s") * info.num_cores + lax.axis_index("c")
        base = wid * per_w

        def body(i, carry):
            off = base + i * chunk
            pltpu.sync_copy(idx_hbm.at[pl.ds(off, chunk)], idx_v)
            pltpu.async_copy(table_hbm.at[idx_v], rows_v, sem).wait()
            pltpu.sync_copy(rows_v, out_hbm.at[pl.ds(off, chunk)])
            return carry

        lax.fori_loop(0, n_chunks, body, 0)

    return k


def _stats_body(k_neigh, c, g_ref, p_ref, b1_ref, out_ref, acc_ref):
    i = pl.program_id(0)
    rk, _ = g_ref.shape
    r = rk // k_neigh
    h = g_ref[:, :c].reshape(r, k_neigh, c) - p_ref[...][:, None, :] + b1_ref[0][None, None, :]

    @pl.when(i == 0)
    def _():
        acc_ref[...] = jnp.zeros_like(acc_ref)

    acc_ref[0, :] += jnp.sum(h, axis=(0, 1))
    acc_ref[1, :] += jnp.sum(h * h, axis=(0, 1))

    @pl.when(i == pl.num_programs(0) - 1)
    def _():
        out_ref[...] = acc_ref[...]


def _stats(g, p_flat, bt1, k_neigh, row_blk):
    bn, c = p_flat.shape
    grid = (bn // row_blk,)
    return pl.pallas_call(
        functools.partial(_stats_body, k_neigh, c),
        grid=grid,
        in_specs=[
            pl.BlockSpec((row_blk * k_neigh, g.shape[1]), lambda i: (i, 0)),
            pl.BlockSpec((row_blk, c), lambda i: (i, 0)),
            pl.BlockSpec((1, c), lambda i: (0, 0)),
        ],
        out_specs=pl.BlockSpec((2, c), lambda i: (0, 0)),
        out_shape=jax.ShapeDtypeStruct((2, c), jnp.float32),
        scratch_shapes=[pltpu.VMEM((2, c), jnp.float32)],
    )(g, p_flat, bt1.reshape(1, c))


def _head_body(k_neigh, c, ce, n_edges, g_ref, p_ref, x_ref, e_ref, st_ref,
               b1_ref, g1_ref, be1_ref, w2_ref, b2_ref, wpx_ref, bpx_ref,
               wte_ref, bte_ref, wpe_ref, bpe_ref, pos_out_ref, ene_out_ref):
    rk, _ = g_ref.shape
    r = rk // k_neigh
    mu = st_ref[0, :] / n_edges
    var = st_ref[1, :] / n_edges - mu * mu
    rstd = lax.rsqrt(var + 1e-5)
    scale = rstd * g1_ref[0]
    shift = be1_ref[0] + (b1_ref[0] - mu) * scale

    h1 = g_ref[:, :c].reshape(r, k_neigh, c) - p_ref[...][:, None, :]
    hn = h1 * scale[None, None, :] + shift[None, None, :]
    relu = jnp.maximum(hn, 0.0).reshape(rk, c)
    h2 = jnp.dot(relu, w2_ref[...], preferred_element_type=jnp.float32)
    hx = jnp.max(h2.reshape(r, k_neigh, c), axis=1)
    skip_x = jnp.dot(x_ref[...], wpx_ref[...], preferred_element_type=jnp.float32)
    pos_out_ref[...] = hx + b2_ref[0][None, :] + skip_x + bpx_ref[0][None, :]

    esum = jnp.sum(g_ref[:, c:c + ce].reshape(r, k_neigh, ce), axis=1)
    de = esum * (1.0 / k_neigh) - e_ref[...]
    he = jnp.dot(de, wte_ref[...], preferred_element_type=jnp.float32)
    skip_e = jnp.dot(e_ref[...], wpe_ref[...], preferred_element_type=jnp.float32)
    ene_out_ref[...] = he + bte_ref[0][None, :] + skip_e + bpe_ref[0][None, :]


def _head(g, p_flat, x_flat, e_flat, stats, b1, g1, be1, w2, b2, wpx, bpx,
          wte, bte, wpe, bpe, k_neigh, row_blk):
    bn, c = p_flat.shape
    ce = e_flat.shape[1]
    d = g.shape[1]
    n_edges = float(g.shape[0])
    grid = (bn // row_blk,)
    vec = lambda v: v.reshape(1, -1)
    full = lambda shape: pl.BlockSpec(shape, lambda i: tuple(0 for _ in shape))
    return pl.pallas_call(
        functools.partial(_head_body, k_neigh, c, ce, n_edges),
        grid=grid,
        in_specs=[
            pl.BlockSpec((row_blk * k_neigh, d), lambda i: (i, 0)),
            pl.BlockSpec((row_blk, c), lambda i: (i, 0)),
            pl.BlockSpec((row_blk, c), lambda i: (i, 0)),
            pl.BlockSpec((row_blk, ce), lambda i: (i, 0)),
            full((2, c)),
            full((1, c)), full((1, c)), full((1, c)),
            full((c, c)), full((1, c)),
            full((c, c)), full((1, c)),
            full((ce, ce)), full((1, ce)),
            full((ce, ce)), full((1, ce)),
        ],
        out_specs=[
            pl.BlockSpec((row_blk, c), lambda i: (i, 0)),
            pl.BlockSpec((row_blk, ce), lambda i: (i, 0)),
        ],
        out_shape=[
            jax.ShapeDtypeStruct((bn, c), jnp.float32),
            jax.ShapeDtypeStruct((bn, ce), jnp.float32),
        ],
    )(g, p_flat, x_flat, e_flat, stats, vec(b1), vec(g1), vec(be1), w2, vec(b2),
      wpx, vec(bpx), wte, vec(bte), wpe, vec(bpe))


def kernel(pos_feat, ene_feat, Wt1, bt1, g1, be1, Wt2, bt2, Wpx, bpx, Wte, bte, Wpe, bpe):
    b, n, c = pos_feat.shape
    ce = ene_feat.shape[2]
    k_neigh = min(20, n - 1)

    idx128, p = _knn(pos_feat, Wt1, k_neigh, row_blk=128)
    gidx = idx128.reshape(b * n, _LANES)[:, :k_neigh].reshape(-1)

    d_pad = 128
    pad = jnp.zeros((b * n, d_pad - c - ce), jnp.float32)
    table = jnp.concatenate(
        [p.reshape(b * n, c), ene_feat.reshape(b * n, ce), pad], axis=1
    )
    g = _make_sc_gather(gidx.shape[0], d_pad, chunk=128)(table, gidx)

    p_flat = p.reshape(b * n, c)
    stats = _stats(g, p_flat, bt1, k_neigh, row_blk=512)
    pos_o, ene_o = _head(
        g, p_flat, pos_feat.reshape(b * n, c), ene_feat.reshape(b * n, ce),
        stats, bt1, g1, be1, Wt2, bt2, Wpx, bpx, Wte, bte, Wpe, bpe,
        k_neigh, row_blk=256,
    )
    return (pos_o.reshape(b, n, c), ene_o.reshape(b, n, ce))


# speedup vs baseline: 1.4990x; 1.4990x over previous
import functools

import jax
import jax.numpy as jnp
from jax import lax
from jax.experimental import pallas as pl
from jax.experimental.pallas import tpu as pltpu
from jax.experimental.pallas import tpu_sc as plsc

_LANES = 128


def _knn_body(k_neigh, cc, xall_ref, xblk_ref, w1_ref, idx_ref, p_ref, s_ref,
              sq_ref, col_ref):
    b = pl.program_id(0)
    rblk = pl.program_id(1)
    xblk = xblk_ref[0]
    r, n = s_ref.shape
    n_cc = n // cc
    row_g = rblk * r + lax.broadcasted_iota(jnp.int32, (r, cc), 0)
    inf = jnp.float32(jnp.inf)

    @pl.when(rblk == 0)
    def _():
        for j in range(n_cc):
            xc = xall_ref[0, pl.ds(j * cc, cc), :]
            sq_ref[pl.ds(j * cc, cc), :] = jnp.sum(xc * xc, axis=1, keepdims=True)

    @pl.when((b == 0) & (rblk == 0))
    def _():
        for j in range(n_cc):
            col_ref[:, pl.ds(j * cc, cc)] = (
                jnp.float32(j * cc)
                + lax.broadcasted_iota(jnp.int32, (r, cc), 1).astype(jnp.float32)
            )

    xblk2 = -2.0 * xblk
    ones_col = jnp.ones((r, 1), jnp.float32)
    big = jnp.float32(3.0e38)

    for j in range(n_cc):
        xc = xall_ref[0, pl.ds(j * cc, cc), :]
        sq = sq_ref[pl.ds(j * cc, cc), :]
        dot = lax.dot_general(
            xblk2, xc, (((1,), (1,)), ((), ())), preferred_element_type=jnp.float32
        )
        bcast = lax.dot_general(
            ones_col, sq, (((1,), (1,)), ((), ())),
            preferred_element_type=jnp.float32,
            precision=lax.Precision.HIGHEST,
        )
        sv = bcast + dot
        col = j * cc + lax.broadcasted_iota(jnp.int32, (r, cc), 1)
        s_ref[:, pl.ds(j * cc, cc)] = jnp.where(col == row_g, big, sv)

    lane = lax.broadcasted_iota(jnp.int32, (r, _LANES), 1)
    base = b * n
    lanes_per_chunk = cc // _LANES

    def step(t, carry):
        amin_prev, acc = carry

        m128 = jnp.full((r, _LANES), big)
        for j in range(n_cc):
            sv = s_ref[:, pl.ds(j * cc, cc)]
            cf = col_ref[:, pl.ds(j * cc, cc)]
            sv = jnp.where(cf == amin_prev, big, sv)
            s_ref[:, pl.ds(j * cc, cc)] = sv
            folded = jnp.min(sv.reshape(r, lanes_per_chunk, _LANES), axis=1)
            m128 = jnp.minimum(m128, folded)
        m = jnp.min(m128, axis=1, keepdims=True)

        a128 = jnp.full((r, _LANES), big)
        for j in range(n_cc):
            sv = s_ref[:, pl.ds(j * cc, cc)]
            cf = col_ref[:, pl.ds(j * cc, cc)]
            cand = jnp.where(sv <= m, cf, big)
            folded = jnp.min(cand.reshape(r, lanes_per_chunk, _LANES), axis=1)
            a128 = jnp.minimum(a128, folded)
        amin = jnp.min(a128, axis=1, keepdims=True)
        acc = jnp.where(lane == t, amin.astype(jnp.int32) + base, acc)
        return amin, acc

    _, acc = lax.fori_loop(
        0, k_neigh, step,
        (jnp.full((r, 1), -1.0, jnp.float32), jnp.zeros((r, _LANES), jnp.int32)),
    )
    idx_ref[0] = acc
    p_ref[0] = jnp.dot(xblk, w1_ref[...], preferred_element_type=jnp.float32)


def _knn(pos, w1, k_neigh, row_blk, col_chunk=512):
    b, n, c = pos.shape
    grid = (b, n // row_blk)
    return pl.pallas_call(
        functools.partial(_knn_body, k_neigh, col_chunk),
        grid=grid,
        in_specs=[
            pl.BlockSpec((1, n, c), lambda i, j: (i, 0, 0)),
            pl.BlockSpec((1, row_blk, c), lambda i, j: (i, j, 0)),
            pl.BlockSpec((c, c), lambda i, j: (0, 0)),
        ],
        out_specs=[
            pl.BlockSpec((1, row_blk, _LANES), lambda i, j: (i, j, 0)),
            pl.BlockSpec((1, row_blk, c), lambda i, j: (i, j, 0)),
        ],
        out_shape=[
            jax.ShapeDtypeStruct((b, n, _LANES), jnp.int32),
            jax.ShapeDtypeStruct((b, n, c), jnp.float32),
        ],
        scratch_shapes=[
            pltpu.VMEM((row_blk, n), jnp.float32),
            pltpu.VMEM((n, 1), jnp.float32),
            pltpu.VMEM((row_blk, n), jnp.float32),
        ],
    )(pos, pos, w1)


def _make_sc_gather(n_idx, d, chunk):
    info = plsc.get_sparse_core_info()
    nw = info.num_cores * info.num_subcores
    per_w = n_idx // nw
    n_chunks = per_w // chunk
    mesh = plsc.VectorSubcoreMesh(core_axis_name="c", subcore_axis_name="s")

    @functools.partial(
        pl.kernel,
        out_type=jax.ShapeDtypeStruct((n_idx, d), jnp.float32),
        mesh=mesh,
        scratch_types=[
            pltpu.VMEM((chunk,), jnp.int32),
            pltpu.VMEM((chunk, d), jnp.float32),
            pltpu.SemaphoreType.DMA,
        ],
    )
    def k(table_hbm, idx_hbm, out_hbm, idx_v, rows_v, sem):
        wid = lax.axis_index("s") * info.num_cores + lax.axis_index("c")
        base = wid * per_w

        def body(i, carry):
            off = base + i * chunk
            pltpu.sync_copy(idx_hbm.at[pl.ds(off, chunk)], idx_v)
            pltpu.async_copy(table_hbm.at[idx_v], rows_v, sem).wait()
            pltpu.sync_copy(rows_v, out_hbm.at[pl.ds(off, chunk)])
            return carry

        lax.fori_loop(0, n_chunks, body, 0)

    return k


def _stats_body(k_neigh, c, g_ref, p_ref, b1_ref, out_ref, acc_ref):
    i = pl.program_id(0)
    rk, _ = g_ref.shape
    r = rk // k_neigh
    h = g_ref[:, :c].reshape(r, k_neigh, c) - p_ref[...][:, None, :] + b1_ref[0][None, None, :]

    @pl.when(i == 0)
    def _():
        acc_ref[...] = jnp.zeros_like(acc_ref)

    acc_ref[0, :] += jnp.sum(h, axis=(0, 1))
    acc_ref[1, :] += jnp.sum(h * h, axis=(0, 1))

    @pl.when(i == pl.num_programs(0) - 1)
    def _():
        out_ref[...] = acc_ref[...]


def _stats(g, p_flat, bt1, k_neigh, row_blk):
    bn, c = p_flat.shape
    grid = (bn // row_blk,)
    return pl.pallas_call(
        functools.partial(_stats_body, k_neigh, c),
        grid=grid,
        in_specs=[
            pl.BlockSpec((row_blk * k_neigh, g.shape[1]), lambda i: (i, 0)),
            pl.BlockSpec((row_blk, c), lambda i: (i, 0)),
            pl.BlockSpec((1, c), lambda i: (0, 0)),
        ],
        out_specs=pl.BlockSpec((2, c), lambda i: (0, 0)),
        out_shape=jax.ShapeDtypeStruct((2, c), jnp.float32),
        scratch_shapes=[pltpu.VMEM((2, c), jnp.float32)],
    )(g, p_flat, bt1.reshape(1, c))


def _head_body(k_neigh, c, ce, n_edges, g_ref, p_ref, x_ref, e_ref, st_ref,
               b1_ref, g1_ref, be1_ref, w2_ref, b2_ref, wpx_ref, bpx_ref,
               wte_ref, bte_ref, wpe_ref, bpe_ref, pos_out_ref, ene_out_ref):
    rk, _ = g_ref.shape
    r = rk // k_neigh
    mu = st_ref[0, :] / n_edges
    var = st_ref[1, :] / n_edges - mu * mu
    rstd = lax.rsqrt(var + 1e-5)
    scale = rstd * g1_ref[0]
    shift = be1_ref[0] + (b1_ref[0] - mu) * scale

    h1 = g_ref[:, :c].reshape(r, k_neigh, c) - p_ref[...][:, None, :]
    hn = h1 * scale[None, None, :] + shift[None, None, :]
    relu = jnp.maximum(hn, 0.0).reshape(rk, c)
    h2 = jnp.dot(relu, w2_ref[...], preferred_element_type=jnp.float32)
    hx = jnp.max(h2.reshape(r, k_neigh, c), axis=1)
    skip_x = jnp.dot(x_ref[...], wpx_ref[...], preferred_element_type=jnp.float32)
    pos_out_ref[...] = hx + b2_ref[0][None, :] + skip_x + bpx_ref[0][None, :]

    esum = jnp.sum(g_ref[:, c:c + ce].reshape(r, k_neigh, ce), axis=1)
    de = esum * (1.0 / k_neigh) - e_ref[...]
    he = jnp.dot(de, wte_ref[...], preferred_element_type=jnp.float32)
    skip_e = jnp.dot(e_ref[...], wpe_ref[...], preferred_element_type=jnp.float32)
    ene_out_ref[...] = he + bte_ref[0][None, :] + skip_e + bpe_ref[0][None, :]


def _head(g, p_flat, x_flat, e_flat, stats, b1, g1, be1, w2, b2, wpx, bpx,
          wte, bte, wpe, bpe, k_neigh, row_blk):
    bn, c = p_flat.shape
    ce = e_flat.shape[1]
    d = g.shape[1]
    n_edges = float(g.shape[0])
    grid = (bn // row_blk,)
    vec = lambda v: v.reshape(1, -1)
    full = lambda shape: pl.BlockSpec(shape, lambda i: tuple(0 for _ in shape))
    return pl.pallas_call(
        functools.partial(_head_body, k_neigh, c, ce, n_edges),
        grid=grid,
        in_specs=[
            pl.BlockSpec((row_blk * k_neigh, d), lambda i: (i, 0)),
            pl.BlockSpec((row_blk, c), lambda i: (i, 0)),
            pl.BlockSpec((row_blk, c), lambda i: (i, 0)),
            pl.BlockSpec((row_blk, ce), lambda i: (i, 0)),
            full((2, c)),
            full((1, c)), full((1, c)), full((1, c)),
            full((c, c)), full((1, c)),
            full((c, c)), full((1, c)),
            full((ce, ce)), full((1, ce)),
            full((ce, ce)), full((1, ce)),
        ],
        out_specs=[
            pl.BlockSpec((row_blk, c), lambda i: (i, 0)),
            pl.BlockSpec((row_blk, ce), lambda i: (i, 0)),
        ],
        out_shape=[
            jax.ShapeDtypeStruct((bn, c), jnp.float32),
            jax.ShapeDtypeStruct((bn, ce), jnp.float32),
        ],
    )(g, p_flat, x_flat, e_flat, stats, vec(b1), vec(g1), vec(be1), w2, vec(b2),
      wpx, vec(bpx), wte, vec(bte), wpe, vec(bpe))


def kernel(pos_feat, ene_feat, Wt1, bt1, g1, be1, Wt2, bt2, Wpx, bpx, Wte, bte, Wpe, bpe):
    b, n, c = pos_feat.shape
    ce = ene_feat.shape[2]
    k_neigh = min(20, n - 1)

    idx128, p = _knn(pos_feat, Wt1, k_neigh, row_blk=128)
    gidx = idx128.reshape(b * n, _LANES)[:, :k_neigh].reshape(-1)

    d_pad = 128
    pad = jnp.zeros((b * n, d_pad - c - ce), jnp.float32)
    table = jnp.concatenate(
        [p.reshape(b * n, c), ene_feat.reshape(b * n, ce), pad], axis=1
    )
    g = _make_sc_gather(gidx.shape[0], d_pad, chunk=128)(table, gidx)

    p_flat = p.reshape(b * n, c)
    stats = _stats(g, p_flat, bt1, k_neigh, row_blk=512)
    pos_o, ene_o = _head(
        g, p_flat, pos_feat.reshape(b * n, c), ene_feat.reshape(b * n, ce),
        stats, bt1, g1, be1, Wt2, bt2, Wpx, bpx, Wte, bte, Wpe, bpe,
        k_neigh, row_blk=256,
    )
    return (pos_o.reshape(b, n, c), ene_o.reshape(b, n, ce))


# speedup vs baseline: 5.8163x; 3.8802x over previous
import functools

import jax
import jax.numpy as jnp
from jax import lax
from jax.experimental import pallas as pl
from jax.experimental.pallas import tpu as pltpu
from jax.experimental.pallas import tpu_sc as plsc

_LANES = 128


def _knn_body(k_neigh, cc, xall_ref, xblk_ref, w1_ref, idx_ref, p_ref, s_ref,
              sq_ref, col_ref):
    b = pl.program_id(0)
    rblk = pl.program_id(1)
    xblk = xblk_ref[0]
    r, n = s_ref.shape
    n_cc = n // cc
    row_g = rblk * r + lax.broadcasted_iota(jnp.int32, (r, cc), 0)
    inf = jnp.float32(jnp.inf)

    @pl.when(rblk == 0)
    def _():
        for j in range(n_cc):
            xc = xall_ref[0, pl.ds(j * cc, cc), :]
            sq_ref[pl.ds(j * cc, cc), :] = jnp.sum(xc * xc, axis=1, keepdims=True)

    @pl.when((b == 0) & (rblk == 0))
    def _():
        for j in range(n_cc):
            col_ref[:, pl.ds(j * cc, cc)] = (
                jnp.float32(j * cc)
                + lax.broadcasted_iota(jnp.int32, (r, cc), 1).astype(jnp.float32)
            )

    xblk2 = -2.0 * xblk
    ones_col = jnp.ones((r, 1), jnp.float32)
    big = jnp.float32(3.0e38)

    for j in range(n_cc):
        xc = xall_ref[0, pl.ds(j * cc, cc), :]
        sq = sq_ref[pl.ds(j * cc, cc), :]
        dot = lax.dot_general(
            xblk2, xc, (((1,), (1,)), ((), ())), preferred_element_type=jnp.float32
        )
        bcast = lax.dot_general(
            ones_col, sq, (((1,), (1,)), ((), ())),
            preferred_element_type=jnp.float32,
            precision=lax.Precision.HIGHEST,
        )
        sv = bcast + dot
        col = j * cc + lax.broadcasted_iota(jnp.int32, (r, cc), 1)
        s_ref[:, pl.ds(j * cc, cc)] = jnp.where(col == row_g, big, sv)

    lane = lax.broadcasted_iota(jnp.int32, (r, _LANES), 1)
    base = b * n
    lanes_per_chunk = cc // _LANES

    def step(t, carry):
        amin_prev, acc = carry

        m128 = jnp.full((r, _LANES), big)
        for j in range(n // _LANES):
            sv = s_ref[:, pl.ds(j * _LANES, _LANES)]
            cf = col_ref[:, pl.ds(j * _LANES, _LANES)]
            sv = jnp.where(cf == amin_prev, big, sv)
            s_ref[:, pl.ds(j * _LANES, _LANES)] = sv
            m128 = jnp.minimum(m128, sv)
        m = jnp.min(m128, axis=1, keepdims=True)

        a128 = jnp.full((r, _LANES), big)
        for j in range(n // _LANES):
            sv = s_ref[:, pl.ds(j * _LANES, _LANES)]
            cf = col_ref[:, pl.ds(j * _LANES, _LANES)]
            a128 = jnp.minimum(a128, jnp.where(sv <= m, cf, big))
        amin = jnp.min(a128, axis=1, keepdims=True)
        acc = jnp.where(lane == t, amin.astype(jnp.int32) + base, acc)
        return amin, acc

    _, acc = lax.fori_loop(
        0, k_neigh, step,
        (jnp.full((r, 1), -1.0, jnp.float32), jnp.zeros((r, _LANES), jnp.int32)),
    )
    idx_ref[0] = acc
    p_ref[0] = jnp.dot(xblk, w1_ref[...], preferred_element_type=jnp.float32)


def _knn(pos, w1, k_neigh, row_blk, col_chunk=512):
    b, n, c = pos.shape
    grid = (b, n // row_blk)
    return pl.pallas_call(
        functools.partial(_knn_body, k_neigh, col_chunk),
        grid=grid,
        in_specs=[
            pl.BlockSpec((1, n, c), lambda i, j: (i, 0, 0)),
            pl.BlockSpec((1, row_blk, c), lambda i, j: (i, j, 0)),
            pl.BlockSpec((c, c), lambda i, j: (0, 0)),
        ],
        out_specs=[
            pl.BlockSpec((1, row_blk, _LANES), lambda i, j: (i, j, 0)),
            pl.BlockSpec((1, row_blk, c), lambda i, j: (i, j, 0)),
        ],
        out_shape=[
            jax.ShapeDtypeStruct((b, n, _LANES), jnp.int32),
            jax.ShapeDtypeStruct((b, n, c), jnp.float32),
        ],
        scratch_shapes=[
            pltpu.VMEM((row_blk, n), jnp.float32),
            pltpu.VMEM((n, 1), jnp.float32),
            pltpu.VMEM((row_blk, n), jnp.float32),
        ],
    )(pos, pos, w1)


def _make_sc_gather(n_idx, d, chunk):
    info = plsc.get_sparse_core_info()
    nw = info.num_cores * info.num_subcores
    per_w = n_idx // nw
    n_chunks = per_w // chunk
    mesh = plsc.VectorSubcoreMesh(core_axis_name="c", subcore_axis_name="s")

    @functools.partial(
        pl.kernel,
        out_type=jax.ShapeDtypeStruct((n_idx, d), jnp.float32),
        mesh=mesh,
        scratch_types=[
            pltpu.VMEM((chunk,), jnp.int32),
            pltpu.VMEM((chunk, d), jnp.float32),
            pltpu.SemaphoreType.DMA,
        ],
    )
    def k(table_hbm, idx_hbm, out_hbm, idx_v, rows_v, sem):
        wid = lax.axis_index("s") * info.num_cores + lax.axis_index("c")
        base = wid * per_w

        def body(i, carry):
            off = base + i * chunk
            pltpu.sync_copy(idx_hbm.at[pl.ds(off, chunk)], idx_v)
            pltpu.async_copy(table_hbm.at[idx_v], rows_v, sem).wait()
            pltpu.sync_copy(rows_v, out_hbm.at[pl.ds(off, chunk)])
            return carry

        lax.fori_loop(0, n_chunks, body, 0)

    return k


def _stats_body(k_neigh, c, g_ref, p_ref, b1_ref, out_ref, acc_ref):
    i = pl.program_id(0)
    rk, _ = g_ref.shape
    r = rk // k_neigh
    h = g_ref[:, :c].reshape(r, k_neigh, c) - p_ref[...][:, None, :] + b1_ref[0][None, None, :]

    @pl.when(i == 0)
    def _():
        acc_ref[...] = jnp.zeros_like(acc_ref)

    acc_ref[0, :] += jnp.sum(h, axis=(0, 1))
    acc_ref[1, :] += jnp.sum(h * h, axis=(0, 1))

    @pl.when(i == pl.num_programs(0) - 1)
    def _():
        out_ref[...] = acc_ref[...]


def _stats(g, p_flat, bt1, k_neigh, row_blk):
    bn, c = p_flat.shape
    grid = (bn // row_blk,)
    return pl.pallas_call(
        functools.partial(_stats_body, k_neigh, c),
        grid=grid,
        in_specs=[
            pl.BlockSpec((row_blk * k_neigh, g.shape[1]), lambda i: (i, 0)),
            pl.BlockSpec((row_blk, c), lambda i: (i, 0)),
            pl.BlockSpec((1, c), lambda i: (0, 0)),
        ],
        out_specs=pl.BlockSpec((2, c), lambda i: (0, 0)),
        out_shape=jax.ShapeDtypeStruct((2, c), jnp.float32),
        scratch_shapes=[pltpu.VMEM((2, c), jnp.float32)],
    )(g, p_flat, bt1.reshape(1, c))


def _head_body(k_neigh, c, ce, n_edges, g_ref, p_ref, x_ref, e_ref, st_ref,
               b1_ref, g1_ref, be1_ref, w2_ref, b2_ref, wpx_ref, bpx_ref,
               wte_ref, bte_ref, wpe_ref, bpe_ref, pos_out_ref, ene_out_ref):
    rk, _ = g_ref.shape
    r = rk // k_neigh
    mu = st_ref[0, :] / n_edges
    var = st_ref[1, :] / n_edges - mu * mu
    rstd = lax.rsqrt(var + 1e-5)
    scale = rstd * g1_ref[0]
    shift = be1_ref[0] + (b1_ref[0] - mu) * scale

    h1 = g_ref[:, :c].reshape(r, k_neigh, c) - p_ref[...][:, None, :]
    hn = h1 * scale[None, None, :] + shift[None, None, :]
    relu = jnp.maximum(hn, 0.0).reshape(rk, c)
    h2 = jnp.dot(relu, w2_ref[...], preferred_element_type=jnp.float32)
    hx = jnp.max(h2.reshape(r, k_neigh, c), axis=1)
    skip_x = jnp.dot(x_ref[...], wpx_ref[...], preferred_element_type=jnp.float32)
    pos_out_ref[...] = hx + b2_ref[0][None, :] + skip_x + bpx_ref[0][None, :]

    esum = jnp.sum(g_ref[:, c:c + ce].reshape(r, k_neigh, ce), axis=1)
    de = esum * (1.0 / k_neigh) - e_ref[...]
    he = jnp.dot(de, wte_ref[...], preferred_element_type=jnp.float32)
    skip_e = jnp.dot(e_ref[...], wpe_ref[...], preferred_element_type=jnp.float32)
    ene_out_ref[...] = he + bte_ref[0][None, :] + skip_e + bpe_ref[0][None, :]


def _head(g, p_flat, x_flat, e_flat, stats, b1, g1, be1, w2, b2, wpx, bpx,
          wte, bte, wpe, bpe, k_neigh, row_blk):
    bn, c = p_flat.shape
    ce = e_flat.shape[1]
    d = g.shape[1]
    n_edges = float(g.shape[0])
    grid = (bn // row_blk,)
    vec = lambda v: v.reshape(1, -1)
    full = lambda shape: pl.BlockSpec(shape, lambda i: tuple(0 for _ in shape))
    return pl.pallas_call(
        functools.partial(_head_body, k_neigh, c, ce, n_edges),
        grid=grid,
        in_specs=[
            pl.BlockSpec((row_blk * k_neigh, d), lambda i: (i, 0)),
            pl.BlockSpec((row_blk, c), lambda i: (i, 0)),
            pl.BlockSpec((row_blk, c), lambda i: (i, 0)),
            pl.BlockSpec((row_blk, ce), lambda i: (i, 0)),
            full((2, c)),
            full((1, c)), full((1, c)), full((1, c)),
            full((c, c)), full((1, c)),
            full((c, c)), full((1, c)),
            full((ce, ce)), full((1, ce)),
            full((ce, ce)), full((1, ce)),
        ],
        out_specs=[
            pl.BlockSpec((row_blk, c), lambda i: (i, 0)),
            pl.BlockSpec((row_blk, ce), lambda i: (i, 0)),
        ],
        out_shape=[
            jax.ShapeDtypeStruct((bn, c), jnp.float32),
            jax.ShapeDtypeStruct((bn, ce), jnp.float32),
        ],
    )(g, p_flat, x_flat, e_flat, stats, vec(b1), vec(g1), vec(be1), w2, vec(b2),
      wpx, vec(bpx), wte, vec(bte), wpe, vec(bpe))


def kernel(pos_feat, ene_feat, Wt1, bt1, g1, be1, Wt2, bt2, Wpx, bpx, Wte, bte, Wpe, bpe):
    b, n, c = pos_feat.shape
    ce = ene_feat.shape[2]
    k_neigh = min(20, n - 1)

    idx128, p = _knn(pos_feat, Wt1, k_neigh, row_blk=128)
    gidx = idx128.reshape(b * n, _LANES)[:, :k_neigh].reshape(-1)

    d_pad = 128
    pad = jnp.zeros((b * n, d_pad - c - ce), jnp.float32)
    table = jnp.concatenate(
        [p.reshape(b * n, c), ene_feat.reshape(b * n, ce), pad], axis=1
    )
    g = _make_sc_gather(gidx.shape[0], d_pad, chunk=128)(table, gidx)

    p_flat = p.reshape(b * n, c)
    stats = _stats(g, p_flat, bt1, k_neigh, row_blk=512)
    pos_o, ene_o = _head(
        g, p_flat, pos_feat.reshape(b * n, c), ene_feat.reshape(b * n, ce),
        stats, bt1, g1, be1, Wt2, bt2, Wpx, bpx, Wte, bte, Wpe, bpe,
        k_neigh, row_blk=256,
    )
    return (pos_o.reshape(b, n, c), ene_o.reshape(b, n, ce))
